# baseline (speedup 1.0000x reference)
import functools

import jax
import jax.numpy as jnp
from jax import lax
from jax.experimental import pallas as pl
from jax.experimental.pallas import tpu as pltpu
from jax.experimental.pallas import tpu_sc as plsc

N_ROWS = 16384
D = 128
N_IDS = 1024
NC = 2
NS = 16
NW = NC * NS
ROWS_PER_W = N_ROWS // NW
CHUNK = 128
CHUNKS_PER_W = ROWS_PER_W // CHUNK
IDS_PER_TILE = N_IDS // NS

_mesh = plsc.VectorSubcoreMesh(core_axis_name="c", subcore_axis_name="s",
                               num_cores=NC, num_subcores=NS)
_sc_params = pltpu.CompilerParams(needs_layout_passes=False)


def _nr_rsqrt(x):
    i = plsc.bitcast(x, jnp.int32)
    i = jnp.int32(0x5F3759DF) - lax.shift_right_logical(i, 1)
    y = plsc.bitcast(i, jnp.float32)
    for _ in range(3):
        y = y * (1.5 - 0.5 * x * y * y)
    return y


@functools.partial(
    pl.kernel,
    out_type=(
        jax.ShapeDtypeStruct((NC, N_IDS, D), jnp.float32),
        jax.ShapeDtypeStruct((NC * N_IDS,), jnp.float32),
    ),
    mesh=_mesh,
    compiler_params=_sc_params,
    scratch_types=(
        pltpu.VMEM((CHUNK, D), jnp.float32),
        pltpu.VMEM((CHUNK, D), jnp.float32),
        pltpu.SemaphoreType.DMA,
        pltpu.VMEM((CHUNK,), jnp.int32),
        pltpu.VMEM((CHUNK,), jnp.int32),
        pltpu.VMEM((CHUNK,), jnp.int32),
        pltpu.VMEM((CHUNK,), jnp.int32),
        pltpu.VMEM((N_IDS,), jnp.float32),
        pltpu.VMEM((NS, N_IDS), jnp.float32),
        pltpu.VMEM_SHARED((N_IDS, D), jnp.float32),
        pltpu.VMEM_SHARED((NS, N_IDS), jnp.float32),
    ),
)
def _sc_accumulate(feat_hbm, ids_hbm, z128_hbm,
                   sums_out, counts_out,
                   feat_v0, feat_v1, fsem, ids0, ids1, ids2, ids3, hist, hv,
                   ssum, shist):
    c = lax.axis_index("c")
    s = lax.axis_index("s")
    w = c * NS + s
    base = w * ROWS_PER_W
    ids_refs = (ids0, ids1, ids2, ids3)
    ones = jnp.ones((16,), jnp.float32)
    zero = jnp.zeros((16,), jnp.float32)

    pltpu.sync_copy(z128_hbm.at[pl.ds(s * IDS_PER_TILE, IDS_PER_TILE)],
                    ssum.at[pl.ds(s * IDS_PER_TILE, IDS_PER_TILE)])
    for k in range(N_IDS // 16):
        hist[pl.ds(k * 16, 16)] = zero
    for j in range(CHUNKS_PER_W):
        pltpu.sync_copy(ids_hbm.at[w * CHUNKS_PER_W + j], ids_refs[j])
    fbufs = (feat_v0, feat_v1)
    descs = {}

    def _start(j):
        descs[j] = pltpu.async_copy(
            feat_hbm.at[pl.ds(base + j * CHUNK, CHUNK)], fbufs[j % 2], fsem)

    _start(0)
    plsc.subcore_barrier()

    for j in range(CHUNKS_PER_W):
        descs[j].wait()
        if j + 1 < CHUNKS_PER_W:
            _start(j + 1)
        for g in range(CHUNK // 16):
            plsc.addupdate_scatter(hist, [ids_refs[j][pl.ds(g * 16, 16)]],
                                   ones)
        pltpu.sync_copy(fbufs[j % 2], ssum.at[ids_refs[j]], add=True)
    plsc.subcore_barrier()

    pltpu.sync_copy(ssum.at[pl.ds(s * IDS_PER_TILE, IDS_PER_TILE)],
                    sums_out.at[c, pl.ds(s * IDS_PER_TILE, IDS_PER_TILE)])
    pltpu.sync_copy(hist, shist.at[s])
    plsc.subcore_barrier()
    pltpu.sync_copy(shist, hv)
    for g in range(IDS_PER_TILE // 16):
        cnt16 = jnp.zeros((16,), jnp.float32)
        for t in range(NS):
            cnt16 = cnt16 + hv[t, pl.ds(s * IDS_PER_TILE + g * 16, 16)]
        hist[pl.ds(g * 16, 16)] = cnt16
    pltpu.sync_copy(hist.at[pl.ds(0, IDS_PER_TILE)],
                    counts_out.at[pl.ds(c * N_IDS + s * IDS_PER_TILE,
                                        IDS_PER_TILE)])


@functools.partial(
    pl.kernel,
    out_type=(
        jax.ShapeDtypeStruct((NC, N_IDS, D), jnp.float32),
        jax.ShapeDtypeStruct((N_IDS,), jnp.float32),
    ),
    mesh=_mesh,
    compiler_params=_sc_params,
    scratch_types=(
        pltpu.VMEM((CHUNK, D // 2), jnp.int32),
        pltpu.VMEM((CHUNK, D // 2), jnp.int32),
        pltpu.VMEM((CHUNK, D), jnp.int32),
        pltpu.SemaphoreType.DMA,
        pltpu.VMEM((CHUNK,), jnp.int32),
        pltpu.VMEM((CHUNK,), jnp.int32),
        pltpu.VMEM((CHUNK,), jnp.int32),
        pltpu.VMEM((CHUNK,), jnp.int32),
        pltpu.VMEM((CHUNK, D), jnp.float32),
        pltpu.VMEM((256,), jnp.float32),
        pltpu.VMEM((IDS_PER_TILE, D), jnp.float32),
        pltpu.VMEM((IDS_PER_TILE, D), jnp.float32),
        pltpu.VMEM((IDS_PER_TILE,), jnp.float32),
        pltpu.VMEM((IDS_PER_TILE,), jnp.float32),
        pltpu.VMEM((IDS_PER_TILE * D,), jnp.float32),
        pltpu.VMEM((IDS_PER_TILE, D), jnp.int32),
        pltpu.VMEM((IDS_PER_TILE,), jnp.float32),
        pltpu.VMEM_SHARED((N_IDS, D), jnp.int32),
        pltpu.VMEM_SHARED((N_IDS, D), jnp.float32),
    ),
)
def _sc_norms(feat_hbm, ids_hbm, sums_hbm, cnts_hbm, z128_hbm,
              norm_out, counts_out,
              feat_v0, feat_v1, cent_v, fsem,
              ids0, ids1, ids2, ids3, norm_v, pbuf,
              va, vb, hv0, hv1, cent_f, cent_bi, cnt_v, scent, snorm):
    c = lax.axis_index("c")
    s = lax.axis_index("s")
    w = c * NS + s
    base = w * ROWS_PER_W
    ids_refs = (ids0, ids1, ids2, ids3)
    iota = jnp.arange(16, dtype=jnp.int32)
    zeros_i = jnp.zeros((16,), jnp.int32)
    sl = pl.ds(s * IDS_PER_TILE, IDS_PER_TILE)

    pltpu.sync_copy(z128_hbm.at[sl], snorm.at[sl])
    pltpu.sync_copy(z128_hbm.at[pl.ds(0, CHUNK)], norm_v)
    for j in range(CHUNKS_PER_W):
        pltpu.sync_copy(ids_hbm.at[w * CHUNKS_PER_W + j], ids_refs[j])

    pltpu.sync_copy(sums_hbm.at[0, sl], va)
    pltpu.sync_copy(sums_hbm.at[1, sl], vb)
    pltpu.sync_copy(cnts_hbm.at[pl.ds(s * IDS_PER_TILE, IDS_PER_TILE)], hv0)
    pltpu.sync_copy(cnts_hbm.at[pl.ds(N_IDS + s * IDS_PER_TILE,
                                      IDS_PER_TILE)], hv1)

    def centers_group(g, _):
        cnt16 = hv0[pl.ds(g * 16, 16)] + hv1[pl.ds(g * 16, 16)]
        cnt_v[pl.ds(g * 16, 16)] = cnt16
        inv16 = 1.0 / jnp.maximum(cnt16, 1.0)
        for k in range(16):
            row = g * 16 + k
            inv_s = inv16[k]
            for cc in range(D // 16):
                csl = pl.ds(cc * 16, 16)
                cent_f[pl.ds(row * D + cc * 16, 16)] = (
                    (va[row, csl] + vb[row, csl]) * inv_s)
        return 0

    lax.fori_loop(0, IDS_PER_TILE // 16, centers_group, 0)

    def pack_row(row, _):
        for k in range(D // 32):
            bidx = row * D + k * 32 + 2 * iota
            ce = plsc.load_gather(cent_f, [bidx])
            co = plsc.load_gather(cent_f, [bidx + 1])
            pk = plsc.pack(ce, co, format=plsc.PackFormat.INTERLEAVED)
            cent_bi[row, pl.ds(k * 16, 16)] = plsc.bitcast(pk, jnp.int32)
        return 0

    lax.fori_loop(0, IDS_PER_TILE, pack_row, 0, unroll=2)
    pltpu.sync_copy(cent_bi, scent.at[sl])
    pltpu.sync_copy(cnt_v, counts_out.at[sl])

    fbufs = (feat_v0, feat_v1)
    fdescs = {}

    def _start_f(j):
        fdescs[j] = pltpu.async_copy(
            feat_hbm.at[pl.ds(base + j * CHUNK, CHUNK)], fbufs[j % 2], fsem)

    _start_f(0)
    plsc.subcore_barrier()

    for j in range(CHUNKS_PER_W):
        feat_v = fbufs[j % 2]
        pltpu.sync_copy(scent.at[ids_refs[j]], cent_v)
        fdescs[j].wait()
        if j + 1 < CHUNKS_PER_W:
            _start_f(j + 1)

        def group_body(g, _):
            def row_body(r16, _):
                row = g * 16 + r16
                acc = jnp.zeros((16,), jnp.float32)
                for cc in range(D // 32):
                    csl = pl.ds(cc * 16, 16)
                    f = plsc.bitcast(feat_v[row, csl], jnp.bfloat16)
                    cn = plsc.bitcast(cent_v[row, csl], jnp.bfloat16)
                    d = f - cn
                    da, db = plsc.unpack(d, format=plsc.PackFormat.INTERLEAVED)
                    acc = acc + da * da
                    acc = acc + db * db
                pbuf[pl.ds(r16 * 16, 16)] = acc
                return 0
            lax.fori_loop(0, 16, row_body, 0, unroll=4)
            rs = jnp.zeros((16,), jnp.float32)
            for cc in range(16):
                rs = rs + plsc.load_gather(pbuf, [iota * 16 + cc])
            sq = jnp.maximum(rs, 1e-24)
            norm = sq * _nr_rsqrt(sq)
            plsc.store_scatter(norm_v, [g * 16 + iota, zeros_i], norm)
            return 0

        lax.fori_loop(0, CHUNK // 16, group_body, 0, unroll=2)
        pltpu.sync_copy(norm_v, snorm.at[ids_refs[j]], add=True)
    plsc.subcore_barrier()

    pltpu.sync_copy(snorm.at[sl], norm_out.at[c, sl])


def _tc_finalize_body(c_ref, n_ref, out_ref):
    cnt = c_ref[...]
    nsum = jnp.sum(n_ref[0] + n_ref[1], axis=1, keepdims=True)
    per_id_mean = nsum / jnp.maximum(cnt, 1.0)
    contrib = jnp.where(cnt > 1.0, per_id_mean, 0.0)
    n_unique = jnp.sum((cnt > 0.0).astype(jnp.float32))
    loss = jnp.sum(contrib) / jnp.maximum(n_unique, 1.0)
    out_ref[...] = jnp.broadcast_to(loss, (1, 1))


def kernel(features, identities):
    ids2d = identities.astype(jnp.int32).reshape(NW * CHUNKS_PER_W, CHUNK)
    z128 = jnp.zeros((N_IDS, D), jnp.float32)

    feat_i32 = lax.bitcast_convert_type(
        features.astype(jnp.bfloat16).reshape(N_ROWS, D // 2, 2), jnp.int32)
    sums_p, hist_p = _sc_accumulate(features, ids2d, z128)
    norms_p, counts = _sc_norms(feat_i32, ids2d, sums_p, hist_p, z128)
    loss = pl.pallas_call(
        _tc_finalize_body,
        out_shape=jax.ShapeDtypeStruct((1, 1), jnp.float32),
    )(counts.reshape(N_IDS, 1), norms_p)
    return loss[0, 0]


# speedup vs baseline: 1.3697x; 1.3697x over previous
import functools

import jax
import jax.numpy as jnp
from jax import lax
from jax.experimental import pallas as pl
from jax.experimental.pallas import tpu as pltpu
from jax.experimental.pallas import tpu_sc as plsc

N_ROWS = 16384
D = 128
N_IDS = 1024
NC = 2
NS = 16
NW = NC * NS
ROWS_PER_W = N_ROWS // NW
CHUNK = 128
CHUNKS_PER_W = ROWS_PER_W // CHUNK
IDS_PER_TILE = N_IDS // NS

_mesh = plsc.VectorSubcoreMesh(core_axis_name="c", subcore_axis_name="s",
                               num_cores=NC, num_subcores=NS)
_sc_params = pltpu.CompilerParams(needs_layout_passes=False)


def _nr_rsqrt(x):
    i = plsc.bitcast(x, jnp.int32)
    i = jnp.int32(0x5F3759DF) - lax.shift_right_logical(i, 1)
    y = plsc.bitcast(i, jnp.float32)
    for _ in range(3):
        y = y * (1.5 - 0.5 * x * y * y)
    return y


@functools.partial(
    pl.kernel,
    out_type=(
        jax.ShapeDtypeStruct((NC, N_IDS, D), jnp.float32),
        jax.ShapeDtypeStruct((NC * N_IDS,), jnp.float32),
    ),
    mesh=_mesh,
    compiler_params=_sc_params,
    scratch_types=(
        pltpu.VMEM((CHUNK, D), jnp.float32),
        pltpu.VMEM((CHUNK, D), jnp.float32),
        pltpu.SemaphoreType.DMA,
        pltpu.VMEM((CHUNK,), jnp.int32),
        pltpu.VMEM((CHUNK,), jnp.int32),
        pltpu.VMEM((CHUNK,), jnp.int32),
        pltpu.VMEM((CHUNK,), jnp.int32),
        pltpu.VMEM((N_IDS,), jnp.float32),
        pltpu.VMEM((NS, N_IDS), jnp.float32),
        pltpu.VMEM_SHARED((N_IDS, D), jnp.float32),
        pltpu.VMEM_SHARED((NS, N_IDS), jnp.float32),
    ),
)
def _sc_accumulate(feat_hbm, ids_hbm, z128_hbm,
                   sums_out, counts_out,
                   feat_v0, feat_v1, fsem, ids0, ids1, ids2, ids3, hist, hv,
                   ssum, shist):
    c = lax.axis_index("c")
    s = lax.axis_index("s")
    w = c * NS + s
    base = w * ROWS_PER_W
    ids_refs = (ids0, ids1, ids2, ids3)
    ones = jnp.ones((16,), jnp.float32)
    zero = jnp.zeros((16,), jnp.float32)

    pltpu.sync_copy(z128_hbm.at[pl.ds(s * IDS_PER_TILE, IDS_PER_TILE)],
                    ssum.at[pl.ds(s * IDS_PER_TILE, IDS_PER_TILE)])
    for k in range(N_IDS // 16):
        hist[pl.ds(k * 16, 16)] = zero
    for j in range(CHUNKS_PER_W):
        pltpu.sync_copy(ids_hbm.at[w * CHUNKS_PER_W + j], ids_refs[j])

    fbufs = (feat_v0, feat_v1)
    descs = {}

    def _start(j):
        descs[j] = pltpu.async_copy(
            feat_hbm.at[pl.ds(base + j * CHUNK, CHUNK)], fbufs[j % 2], fsem)

    _start(0)
    plsc.subcore_barrier()

    for j in range(CHUNKS_PER_W):
        descs[j].wait()
        if j + 1 < CHUNKS_PER_W:
            _start(j + 1)
        for g in range(CHUNK // 16):
            plsc.addupdate_scatter(hist, [ids_refs[j][pl.ds(g * 16, 16)]],
                                   ones)
        pltpu.sync_copy(fbufs[j % 2], ssum.at[ids_refs[j]], add=True)
    plsc.subcore_barrier()

    pltpu.sync_copy(ssum.at[pl.ds(s * IDS_PER_TILE, IDS_PER_TILE)],
                    sums_out.at[c, pl.ds(s * IDS_PER_TILE, IDS_PER_TILE)])
    pltpu.sync_copy(hist, shist.at[s])
    plsc.subcore_barrier()
    pltpu.sync_copy(shist, hv)
    for g in range(IDS_PER_TILE // 16):
        cnt16 = jnp.zeros((16,), jnp.float32)
        for t in range(NS):
            cnt16 = cnt16 + hv[t, pl.ds(s * IDS_PER_TILE + g * 16, 16)]
        hist[pl.ds(g * 16, 16)] = cnt16
    pltpu.sync_copy(hist.at[pl.ds(0, IDS_PER_TILE)],
                    counts_out.at[pl.ds(c * N_IDS + s * IDS_PER_TILE,
                                        IDS_PER_TILE)])


@functools.partial(
    pl.kernel,
    out_type=(
        jax.ShapeDtypeStruct((NC, N_IDS, D), jnp.float32),
        jax.ShapeDtypeStruct((N_IDS,), jnp.float32),
    ),
    mesh=_mesh,
    compiler_params=_sc_params,
    scratch_types=(
        pltpu.VMEM((CHUNK, D), jnp.float32),
        pltpu.VMEM((CHUNK, D), jnp.float32),
        pltpu.VMEM((CHUNK, D), jnp.float32),
        pltpu.VMEM((CHUNK, D), jnp.float32),
        pltpu.SemaphoreType.DMA,
        pltpu.SemaphoreType.DMA,
        pltpu.VMEM((CHUNK,), jnp.int32),
        pltpu.VMEM((CHUNK,), jnp.int32),
        pltpu.VMEM((CHUNK,), jnp.int32),
        pltpu.VMEM((CHUNK,), jnp.int32),
        pltpu.VMEM((CHUNK, D), jnp.float32),
        pltpu.VMEM((256,), jnp.float32),
        pltpu.VMEM((IDS_PER_TILE, D), jnp.float32),
        pltpu.VMEM((IDS_PER_TILE, D), jnp.float32),
        pltpu.VMEM((IDS_PER_TILE,), jnp.float32),
        pltpu.VMEM((IDS_PER_TILE,), jnp.float32),
        pltpu.VMEM((IDS_PER_TILE, D), jnp.float32),
        pltpu.VMEM((IDS_PER_TILE,), jnp.float32),
        pltpu.VMEM_SHARED((N_IDS, D), jnp.float32),
        pltpu.VMEM_SHARED((N_IDS, D), jnp.float32),
    ),
)
def _sc_norms(feat_hbm, ids_hbm, sums_hbm, cnts_hbm, z128_hbm,
              norm_out, counts_out,
              feat_v0, feat_v1, cent_v0, cent_v1, fsem, csem,
              ids0, ids1, ids2, ids3, norm_v, pbuf,
              va, vb, hv0, hv1, cent_b, cnt_v, scent, snorm):
    c = lax.axis_index("c")
    s = lax.axis_index("s")
    w = c * NS + s
    base = w * ROWS_PER_W
    ids_refs = (ids0, ids1, ids2, ids3)
    iota = jnp.arange(16, dtype=jnp.int32)
    zeros_i = jnp.zeros((16,), jnp.int32)
    sl = pl.ds(s * IDS_PER_TILE, IDS_PER_TILE)

    pltpu.sync_copy(z128_hbm.at[sl], snorm.at[sl])
    pltpu.sync_copy(z128_hbm.at[pl.ds(0, CHUNK)], norm_v)
    for j in range(CHUNKS_PER_W):
        pltpu.sync_copy(ids_hbm.at[w * CHUNKS_PER_W + j], ids_refs[j])

    pltpu.sync_copy(sums_hbm.at[0, sl], va)
    pltpu.sync_copy(sums_hbm.at[1, sl], vb)
    pltpu.sync_copy(cnts_hbm.at[pl.ds(s * IDS_PER_TILE, IDS_PER_TILE)], hv0)
    pltpu.sync_copy(cnts_hbm.at[pl.ds(N_IDS + s * IDS_PER_TILE,
                                      IDS_PER_TILE)], hv1)

    def centers_group(g, _):
        cnt16 = hv0[pl.ds(g * 16, 16)] + hv1[pl.ds(g * 16, 16)]
        cnt_v[pl.ds(g * 16, 16)] = cnt16
        inv16 = 1.0 / jnp.maximum(cnt16, 1.0)
        for k in range(16):
            row = g * 16 + k
            inv_s = inv16[k]
            for cc in range(D // 16):
                csl = pl.ds(cc * 16, 16)
                cent_b[row, csl] = (va[row, csl] + vb[row, csl]) * inv_s
        return 0

    lax.fori_loop(0, IDS_PER_TILE // 16, centers_group, 0)
    pltpu.sync_copy(cent_b, scent.at[sl])
    pltpu.sync_copy(cnt_v, counts_out.at[sl])

    fbufs = (feat_v0, feat_v1)
    cbufs = (cent_v0, cent_v1)
    fdescs, cdescs = {}, {}

    def _start_f(j):
        fdescs[j] = pltpu.async_copy(
            feat_hbm.at[pl.ds(base + j * CHUNK, CHUNK)], fbufs[j % 2], fsem)

    def _start_c(j):
        cdescs[j] = pltpu.async_copy(scent.at[ids_refs[j]], cbufs[j % 2],
                                     csem)

    _start_f(0)
    plsc.subcore_barrier()
    _start_c(0)

    for j in range(CHUNKS_PER_W):
        feat_v = fbufs[j % 2]
        cent_v = cbufs[j % 2]
        fdescs[j].wait()
        cdescs[j].wait()
        if j + 1 < CHUNKS_PER_W:
            _start_f(j + 1)
            _start_c(j + 1)

        def group_body(g, _):
            def row_body(r16, _):
                row = g * 16 + r16
                acc = jnp.zeros((16,), jnp.float32)
                for cc in range(D // 16):
                    d = (feat_v[row, pl.ds(cc * 16, 16)]
                         - cent_v[row, pl.ds(cc * 16, 16)])
                    acc = acc + d * d
                pbuf[pl.ds(r16 * 16, 16)] = acc
                return 0
            lax.fori_loop(0, 16, row_body, 0, unroll=4)
            rs = jnp.zeros((16,), jnp.float32)
            for cc in range(16):
                rs = rs + plsc.load_gather(pbuf, [iota * 16 + cc])
            sq = jnp.maximum(rs, 1e-24)
            norm = sq * _nr_rsqrt(sq)
            plsc.store_scatter(norm_v, [g * 16 + iota, zeros_i], norm)
            return 0

        lax.fori_loop(0, CHUNK // 16, group_body, 0, unroll=2)
        pltpu.sync_copy(norm_v, snorm.at[ids_refs[j]], add=True)
    plsc.subcore_barrier()

    pltpu.sync_copy(snorm.at[sl], norm_out.at[c, sl])


def _tc_finalize_body(c_ref, n_ref, out_ref):
    cnt = c_ref[...]
    nsum = jnp.sum(n_ref[0] + n_ref[1], axis=1, keepdims=True)
    per_id_mean = nsum / jnp.maximum(cnt, 1.0)
    contrib = jnp.where(cnt > 1.0, per_id_mean, 0.0)
    n_unique = jnp.sum((cnt > 0.0).astype(jnp.float32))
    loss = jnp.sum(contrib) / jnp.maximum(n_unique, 1.0)
    out_ref[...] = jnp.broadcast_to(loss, (1, 1))


def kernel(features, identities):
    ids2d = identities.astype(jnp.int32).reshape(NW * CHUNKS_PER_W, CHUNK)
    z128 = jnp.zeros((N_IDS, D), jnp.float32)

    sums_p, hist_p = _sc_accumulate(features, ids2d, z128)
    norms_p, counts = _sc_norms(features, ids2d, sums_p, hist_p, z128)
    loss = pl.pallas_call(
        _tc_finalize_body,
        out_shape=jax.ShapeDtypeStruct((1, 1), jnp.float32),
    )(counts.reshape(N_IDS, 1), norms_p)
    return loss[0, 0]
